# Initial kernel scaffold; baseline (speedup 1.0000x reference)
import jax
import jax.numpy as jnp
from jax.experimental import pallas as pl


def kernel(spec, diffusion_step, cond, params):
    raise NotImplementedError("write your pallas kernel here")


# speedup vs baseline: 1.3575x; 1.3575x over previous
import functools

import jax
import jax.numpy as jnp
import numpy as np
from jax.experimental import pallas as pl
from jax.experimental.pallas import tpu as pltpu

B = 1; T = 1024; IN_DIMS = 128; HID = 256; C = 512; C2 = 1024
L = 6; E = 8; TOPK = 2; INNER = 512; KS = 31
NEG = -3.4e38


def _ln2d(x, g, b):
    m = jnp.mean(x, -1, keepdims=True)
    d = x - m
    v = jnp.mean(d * d, -1, keepdims=True)
    return d * jax.lax.rsqrt(v + 1e-5) * g + b


def _silu(x):
    return x * jax.nn.sigmoid(x)


def _swiglu(h):
    a = h[:, :INNER]
    g = h[:, INNER:]
    return a * _silu(g)


def _dot(a, b):
    return jax.lax.dot_general(a, b, (((1,), (0,)), ((), ())),
                               preferred_element_type=jnp.float32)


def _pre_kernel(spec_t, cond_t, step, emb, in_w, in_b, cond_w, cond_b,
                de_w1, de_b1, de_w2, de_b2, x1_o, x2_o):
    x = _dot(spec_t[...], in_w[...]) + in_b[...]
    x = x + _dot(cond_t[...], cond_w[...]) + cond_b[...]
    e = step[0, 0] * emb[...]
    e = jnp.concatenate([jnp.sin(e), jnp.cos(e)], axis=1)
    h = _dot(e, de_w1[...]) + de_b1[...]
    d = 0.5 * h * (1.0 + jax.lax.erf(h * np.float32(1.0 / np.sqrt(2.0))))
    d = _dot(d, de_w2[...]) + de_b2[...]
    x = x + d
    x1_o[...] = x[:, :C]
    x2_o[...] = x[:, C:]


def _block_kernel(x1, x2, ln_g, ln_b, conv_wt, conv_b,
                  r_w1, r_b1, r_w2, r_b2,
                  e_w1, e_b1, e_w2, e_b2, e_w3, e_b3,
                  se_w1, se_b1, se_w2, se_b2, se_res,
                  out, x_ln_s, xc_s, wfull_s, yacc_s):
    e = pl.program_id(0)

    @pl.when(e == 0)
    def _pre():
        x = _ln2d(x1[...], ln_g[...], ln_b[...])
        x_ln_s[...] = x
        zpad = jnp.zeros((KS // 2, C), jnp.float32)
        xp = jnp.concatenate([zpad, x, zpad], axis=0)
        acc = jnp.broadcast_to(conv_b[...], (T, C))
        for k in range(KS):
            acc = acc + xp[k:k + T, :] * conv_wt[k:k + 1, :]
        xc_s[...] = acc
        rv = _dot(_silu(_dot(acc, r_w1[...]) + r_b1[...]), r_w2[...]) + r_b2[...]
        cols = jax.lax.broadcasted_iota(jnp.int32, (T, E), 1)
        m1 = jnp.max(rv, axis=1, keepdims=True)
        i1 = jnp.min(jnp.where(rv == m1, cols, E), axis=1, keepdims=True)
        rvm = jnp.where(cols == i1, NEG, rv)
        m2 = jnp.max(rvm, axis=1, keepdims=True)
        i2 = jnp.min(jnp.where(rvm == m2, cols, E), axis=1, keepdims=True)
        sel = (cols == i1) | (cols == i2)
        s = jnp.where(sel, jnp.exp(rv - m1), 0.0)
        wfull_s[...] = s / jnp.sum(s, axis=1, keepdims=True)
        yacc_s[...] = jnp.zeros((T, C), jnp.float32)

    xc = xc_s[...]
    h = _swiglu(_dot(xc, e_w1[0]) + e_b1[0])
    h = _swiglu(_dot(h, e_w2[0]) + e_b2[0])
    h = _dot(h, e_w3[0]) + e_b3[0]
    onehot = (jax.lax.broadcasted_iota(jnp.int32, (1, E), 1) == e)
    wcol = jnp.sum(jnp.where(onehot, wfull_s[...], 0.0), axis=1, keepdims=True)
    yacc_s[...] += wcol * h

    @pl.when(e == E - 1)
    def _post():
        y = xc_s[...] + yacc_s[...]
        x = x_ln_s[...]
        pooled = jnp.mean(x * se_res[...] + y, axis=0, keepdims=True)
        g1 = _silu(_dot(pooled, se_w1[...]) + se_b1[...])
        gate = jax.nn.sigmoid(_dot(g1, se_w2[...]) + se_b2[...])
        out[...] = x2[...] + x1[...] + y * gate


def _final_kernel(x1, x2, g1, g2, b1, b2, w_a, w_b, ob, out):
    a = x1[...]
    b = x2[...]
    m = (jnp.sum(a, 1, keepdims=True) + jnp.sum(b, 1, keepdims=True)) / C2
    da = a - m
    db = b - m
    v = (jnp.sum(da * da, 1, keepdims=True) + jnp.sum(db * db, 1, keepdims=True)) / C2
    r = jax.lax.rsqrt(v + 1e-5)
    na = da * r * g1[...] + b1[...]
    nb = db * r * g2[...] + b2[...]
    out[...] = _dot(na, w_a[...]) + _dot(nb, w_b[...]) + ob[...]


def _full(shape):
    nd = len(shape)
    return pl.BlockSpec(shape, lambda e: (0,) * nd)


def _run_block(x1, x2, pl_):
    espec = [
        pl.BlockSpec((1, C, 2 * INNER), lambda e: (e, 0, 0)),
        pl.BlockSpec((1, 1, 2 * INNER), lambda e: (e, 0, 0)),
        pl.BlockSpec((1, INNER, 2 * INNER), lambda e: (e, 0, 0)),
        pl.BlockSpec((1, 1, 2 * INNER), lambda e: (e, 0, 0)),
        pl.BlockSpec((1, INNER, C), lambda e: (e, 0, 0)),
        pl.BlockSpec((1, 1, C), lambda e: (e, 0, 0)),
    ]
    in_specs = [
        _full((T, C)), _full((T, C)),
        _full((1, C)), _full((1, C)),
        _full((KS, C)), _full((1, C)),
        _full((C, C)), _full((1, C)), _full((C, E)), _full((1, E)),
    ] + espec + [
        _full((C, C // 8)), _full((1, C // 8)),
        _full((C // 8, C)), _full((1, C)), _full((1, C)),
    ]
    return pl.pallas_call(
        _block_kernel,
        grid=(E,),
        in_specs=in_specs,
        out_specs=_full((T, C)),
        out_shape=jax.ShapeDtypeStruct((T, C), jnp.float32),
        scratch_shapes=[
            pltpu.VMEM((T, C), jnp.float32),
            pltpu.VMEM((T, C), jnp.float32),
            pltpu.VMEM((T, E), jnp.float32),
            pltpu.VMEM((T, C), jnp.float32),
        ],
    )(x1, x2, *pl_)


def kernel(spec, diffusion_step, cond, params):
    p = params
    spec_t = jnp.transpose(spec[:, 0], (0, 2, 1)).reshape(T, IN_DIMS)
    cond_t = jnp.transpose(cond, (0, 2, 1)).reshape(T, HID)
    step = diffusion_step.reshape(1, 1)
    half = C // 2
    emb = jnp.exp(jnp.arange(half, dtype=jnp.float32)
                  * jnp.float32(-np.log(10000.0) / (half - 1))).reshape(1, half)

    x1, x2 = pl.pallas_call(
        _pre_kernel,
        out_shape=[jax.ShapeDtypeStruct((T, C), jnp.float32),
                   jax.ShapeDtypeStruct((T, C), jnp.float32)],
    )(spec_t, cond_t, step, emb,
      p["in_w"], p["in_b"].reshape(1, C2),
      p["cond_w"], p["cond_b"].reshape(1, C2),
      p["de_w1"], p["de_b1"].reshape(1, 4 * C),
      p["de_w2"], p["de_b2"].reshape(1, C2))

    for l in range(L):
        pl_ = (
            p["ln_g"][l].reshape(1, C), p["ln_b"][l].reshape(1, C),
            jnp.transpose(p["conv_w"][l, :, 0, :], (1, 0)),
            p["conv_b"][l].reshape(1, C),
            p["r_w1"][l], p["r_b1"][l].reshape(1, C),
            p["r_w2"][l], p["r_b2"][l].reshape(1, E),
            p["e_w1"][l], p["e_b1"][l].reshape(E, 1, 2 * INNER),
            p["e_w2"][l], p["e_b2"][l].reshape(E, 1, 2 * INNER),
            p["e_w3"][l], p["e_b3"][l].reshape(E, 1, C),
            p["se_w1"][l], p["se_b1"][l].reshape(1, C // 8),
            p["se_w2"][l], p["se_b2"][l].reshape(1, C),
            p["se_res"][l].reshape(1, C),
        )
        x2 = _run_block(x1, x2, pl_)
        x1, x2 = x2, x1

    out = pl.pallas_call(
        _final_kernel,
        out_shape=jax.ShapeDtypeStruct((T, IN_DIMS), jnp.float32),
    )(x1, x2,
      p["out_ln_g"][:C].reshape(1, C), p["out_ln_g"][C:].reshape(1, C),
      p["out_ln_b"][:C].reshape(1, C), p["out_ln_b"][C:].reshape(1, C),
      p["out_w"][:C], p["out_w"][C:],
      p["out_b"].reshape(1, IN_DIMS))

    return jnp.transpose(out, (1, 0))[None, None, :, :]
